# baseline (speedup 1.0000x reference)
import jax
import jax.numpy as jnp
from jax.experimental import pallas as pl
from jax.experimental.pallas import tpu as pltpu


def _body(x_ref, wg_ref, we_ref, o_ref, w_scr, ca_scr, cb_scr):
    e = pl.program_id(1)
    nE = 8

    @pl.when(e == 0)
    def _():
        xb = x_ref[...]
        logits = jax.lax.dot_general(
            xb, wg_ref[...], (((1,), (1,)), ((), ())),
            preferred_element_type=jnp.float32)
        m = jnp.max(logits, axis=1, keepdims=True)
        s = jnp.exp(logits - m)
        gate = s / jnp.sum(s, axis=1, keepdims=True)
        iota = jax.lax.broadcasted_iota(jnp.int32, gate.shape, 1)
        v1 = jnp.max(gate, axis=1, keepdims=True)
        i1 = jnp.min(jnp.where(gate == v1, iota, nE), axis=1, keepdims=True)
        g2 = jnp.where(iota == i1, -jnp.inf, gate)
        v2 = jnp.max(g2, axis=1, keepdims=True)
        i2 = jnp.min(jnp.where(g2 == v2, iota, nE), axis=1, keepdims=True)
        wsum = v1 + v2 + 1e-9
        w = (jnp.where(iota == i1, v1 / wsum, 0.0)
             + jnp.where(iota == i2, v2 / wsum, 0.0))
        w_scr[...] = w

    @pl.when(e < nE)
    def _():
        contrib = jax.lax.dot_general(
            x_ref[...], we_ref[0], (((1,), (0,)), ((), ())),
            preferred_element_type=jnp.float32)

        @pl.when(e % 2 == 0)
        def _():
            ca_scr[...] = contrib

        @pl.when(e % 2 == 1)
        def _():
            cb_scr[...] = contrib

    @pl.when(e > 0)
    def _():
        ep = e - 1
        wall = w_scr[...]
        eiota = jax.lax.broadcasted_iota(jnp.int32, wall.shape, 1)
        wcol = jnp.sum(jnp.where(eiota == ep, wall, 0.0), axis=1,
                       keepdims=True)

        @pl.when(ep % 2 == 0)
        def _():
            c = ca_scr[...] * wcol

            @pl.when(ep == 0)
            def _():
                o_ref[...] = c

            @pl.when(ep > 0)
            def _():
                o_ref[...] += c

        @pl.when(ep % 2 == 1)
        def _():
            o_ref[...] += cb_scr[...] * wcol


def kernel(x, W_gate, We):
    B, T, D = x.shape
    E = We.shape[0]
    xf = x.reshape(B * T, D)
    BT_BLK = 1024
    grid = (B * T // BT_BLK, E + 1)
    out = pl.pallas_call(
        _body,
        grid=grid,
        in_specs=[
            pl.BlockSpec((BT_BLK, D), lambda i, e: (i, 0)),
            pl.BlockSpec((E, D), lambda i, e: (0, 0)),
            pl.BlockSpec((1, D, D),
                         lambda i, e: (jnp.minimum(e, E - 1), 0, 0)),
        ],
        out_specs=pl.BlockSpec((BT_BLK, D), lambda i, e: (i, 0)),
        out_shape=jax.ShapeDtypeStruct((B * T, D), jnp.float32),
        scratch_shapes=[
            pltpu.VMEM((BT_BLK, E), jnp.float32),
            pltpu.VMEM((BT_BLK, D), jnp.float32),
            pltpu.VMEM((BT_BLK, D), jnp.float32),
        ],
    )(xf, W_gate, We)
    return out.reshape(B, T, D)


# speedup vs baseline: 1.3284x; 1.3284x over previous
import jax
import jax.numpy as jnp
from jax.experimental import pallas as pl
from jax.experimental.pallas import tpu as pltpu


def _moe_dense_body(x_ref, wg_ref, we_ref, o_ref, w_scr):
    e = pl.program_id(1)
    nE = pl.num_programs(1)

    @pl.when(e == 0)
    def _():
        xb = x_ref[...]
        logits = jax.lax.dot_general(
            xb, wg_ref[...], (((1,), (1,)), ((), ())),
            preferred_element_type=jnp.float32)
        m = jnp.max(logits, axis=1, keepdims=True)
        s = jnp.exp(logits - m)
        gate = s / jnp.sum(s, axis=1, keepdims=True)
        iota = jax.lax.broadcasted_iota(jnp.int32, gate.shape, 1)
        v1 = jnp.max(gate, axis=1, keepdims=True)
        i1 = jnp.min(jnp.where(gate == v1, iota, nE), axis=1, keepdims=True)
        g2 = jnp.where(iota == i1, -jnp.inf, gate)
        v2 = jnp.max(g2, axis=1, keepdims=True)
        i2 = jnp.min(jnp.where(g2 == v2, iota, nE), axis=1, keepdims=True)
        wsum = v1 + v2 + 1e-9
        w = (jnp.where(iota == i1, v1 / wsum, 0.0)
             + jnp.where(iota == i2, v2 / wsum, 0.0))
        w_scr[...] = w

    contrib = jax.lax.dot_general(
        x_ref[...], we_ref[0], (((1,), (0,)), ((), ())),
        preferred_element_type=jnp.float32)
    wall = w_scr[...]
    eiota = jax.lax.broadcasted_iota(jnp.int32, wall.shape, 1)
    wcol = jnp.sum(jnp.where(eiota == e, wall, 0.0), axis=1, keepdims=True)
    contrib = contrib * wcol

    @pl.when(e == 0)
    def _():
        o_ref[...] = contrib

    @pl.when(e != 0)
    def _():
        o_ref[...] += contrib


def kernel(x, W_gate, We):
    B, T, D = x.shape
    E = We.shape[0]
    xf = x.reshape(B * T, D)
    BT_BLK = 2048
    grid = (B * T // BT_BLK, E)
    out = pl.pallas_call(
        _moe_dense_body,
        grid=grid,
        in_specs=[
            pl.BlockSpec((BT_BLK, D), lambda i, e: (i, 0)),
            pl.BlockSpec((E, D), lambda i, e: (0, 0)),
            pl.BlockSpec((1, D, D), lambda i, e: (e, 0, 0)),
        ],
        out_specs=pl.BlockSpec((BT_BLK, D), lambda i, e: (i, 0)),
        out_shape=jax.ShapeDtypeStruct((B * T, D), jnp.float32),
        scratch_shapes=[pltpu.VMEM((BT_BLK, E), jnp.float32)],
    )(xf, W_gate, We)
    return out.reshape(B, T, D)
